# baseline (speedup 1.0000x reference)
import functools

import jax
import jax.numpy as jnp
from jax import lax
from jax.experimental import pallas as pl
from jax.experimental.pallas import tpu as pltpu
from jax.experimental.pallas import tpu_sc as plsc

N = 10000
E = 320000
D = 128

NC = 2
NS = 16
NW = NC * NS
BLK = 80
WBLK = E // (NW * BLK)
CH = 25
NCHUNK = WBLK // CH


def _sc_aggregate(values, zeros, edges):
    mesh = plsc.VectorSubcoreMesh(core_axis_name="c", subcore_axis_name="s")

    @functools.partial(
        pl.kernel,
        out_type=jax.ShapeDtypeStruct((NC, N, D), jnp.float32),
        mesh=mesh,
        scratch_types=[
            pltpu.VMEM((CH, BLK), jnp.int32),
            pltpu.VMEM((CH, BLK), jnp.int32),
            pltpu.VMEM((BLK, D), jnp.float32),
            pltpu.VMEM((BLK, D), jnp.float32),
            pltpu.VMEM((BLK, D), jnp.float32),
            pltpu.VMEM_SHARED((N, D), jnp.float32),
            pltpu.SemaphoreType.DMA,
            pltpu.SemaphoreType.DMA,
            pltpu.SemaphoreType.DMA,
            pltpu.SemaphoreType.DMA,
            pltpu.SemaphoreType.DMA,
            pltpu.SemaphoreType.DMA,
        ],
    )
    def agg_kernel(x_hbm, z_hbm, e_hbm, out_hbm,
                   src_v, dst_v, rows0, rows1, rows2, acc_sh,
                   g0, g1, g2, s0, s1, s2):
        cid = lax.axis_index("c")
        sid = lax.axis_index("s")
        wid = sid * NC + cid

        off = pl.multiple_of(sid * 624, 8)

        @pl.when(sid < 15)
        def _():
            pltpu.sync_copy(z_hbm.at[pl.ds(off, 624)],
                            acc_sh.at[pl.ds(off, 624)])

        @pl.when(sid == 15)
        def _():
            pltpu.sync_copy(z_hbm.at[pl.ds(9360, 640)],
                            acc_sh.at[pl.ds(9360, 640)])

        plsc.subcore_barrier()

        rows = (rows0, rows1, rows2)
        gsem = (g0, g1, g2)
        ssem = (s0, s1, s2)
        gwait = [pltpu.make_async_copy(x_hbm.at[src_v.at[0]], rows[k], gsem[k])
                 for k in range(3)]
        swait = [pltpu.make_async_copy(rows[k], acc_sh.at[dst_v.at[0]],
                                       ssem[k]) for k in range(3)]

        def gath(b, k):
            pltpu.async_copy(x_hbm.at[src_v.at[b]], rows[k], gsem[k])

        def scat(b, k):
            del b, k

        class _NoWait:
            def wait(self):
                pass

        swait = [_NoWait(), _NoWait(), _NoWait()]

        @pl.loop(0, NCHUNK)
        def _(c):
            pltpu.sync_copy(e_hbm.at[0, wid, c], src_v)
            pltpu.sync_copy(e_hbm.at[1, wid, c], dst_v)
            gath(0, 0)
            gath(1, 1)
            gwait[0].wait()
            scat(0, 0)
            gath(2, 2)

            @pl.loop(0, (CH - 4) // 3)
            def _(m):
                j = 3 * m + 1
                gwait[1].wait()
                scat(j, 1)
                swait[0].wait()
                gath(j + 2, 0)
                gwait[2].wait()
                scat(j + 1, 2)
                swait[1].wait()
                gath(j + 3, 1)
                gwait[0].wait()
                scat(j + 2, 0)
                swait[2].wait()
                gath(j + 4, 2)

            gwait[1].wait()
            scat(CH - 3, 1)
            swait[0].wait()
            gath(CH - 1, 0)
            gwait[2].wait()
            scat(CH - 2, 2)
            gwait[0].wait()
            scat(CH - 1, 0)
            swait[1].wait()
            swait[2].wait()
            swait[0].wait()

        plsc.subcore_barrier()

        @pl.when(sid < 15)
        def _():
            pltpu.sync_copy(acc_sh.at[pl.ds(off, 624)],
                            out_hbm.at[cid, pl.ds(off, 624)])

        @pl.when(sid == 15)
        def _():
            pltpu.sync_copy(acc_sh.at[pl.ds(9360, 640)],
                            out_hbm.at[cid, pl.ds(9360, 640)])

    return agg_kernel(values, zeros, edges)


ROW_BLK = 1000


def _mlp1_body(x_ref, p_ref, w1_ref, b1_ref, w2_ref, b2_ref, o_ref):
    h = x_ref[...] + p_ref[0] + p_ref[1]
    a = lax.dot_general(h, w1_ref[...], (((1,), (0,)), ((), ())),
                        precision=lax.Precision.DEFAULT,
                        preferred_element_type=jnp.float32)
    a = jnp.maximum(a + b1_ref[...], 0.0)
    hh = lax.dot_general(a, w2_ref[...], (((1,), (0,)), ((), ())),
                         precision=lax.Precision.DEFAULT,
                         preferred_element_type=jnp.float32)
    hh = hh + b2_ref[...]
    o_ref[...] = jnp.where(hh > 0, hh, jnp.exp(hh) - 1.0)


def _mlp2_body(h_ref, q_ref, w3_ref, b3_ref, o_ref):
    h2 = h_ref[...] + q_ref[0] + q_ref[1]
    a = lax.dot_general(h2, w3_ref[...], (((1,), (0,)), ((), ())),
                        precision=lax.Precision.DEFAULT,
                        preferred_element_type=jnp.float32)
    o_ref[...] = jnp.maximum(a + b3_ref[...], 0.0)


def _row_spec():
    return pl.BlockSpec((ROW_BLK, D), lambda i: (i, 0))


def _pair_spec():
    return pl.BlockSpec((NC, ROW_BLK, D), lambda i: (0, i, 0))


def _full_spec(shape):
    return pl.BlockSpec(shape, lambda i: tuple(0 for _ in shape))


def _mlp1(x, p, W1, b1, W2, b2):
    return pl.pallas_call(
        _mlp1_body,
        grid=(N // ROW_BLK,),
        in_specs=[_row_spec(), _pair_spec(),
                  _full_spec((D, D)), _full_spec((1, D)),
                  _full_spec((D, D)), _full_spec((1, D))],
        out_specs=_row_spec(),
        out_shape=jax.ShapeDtypeStruct((N, D), jnp.float32),
        compiler_params=pltpu.CompilerParams(
            dimension_semantics=("parallel",)),
    )(x, p, W1, b1.reshape(1, D), W2, b2.reshape(1, D))


def _mlp2(h, q, W3, b3):
    return pl.pallas_call(
        _mlp2_body,
        grid=(N // ROW_BLK,),
        in_specs=[_row_spec(), _pair_spec(),
                  _full_spec((D, D)), _full_spec((1, D))],
        out_specs=_row_spec(),
        out_shape=jax.ShapeDtypeStruct((N, D), jnp.float32),
        compiler_params=pltpu.CompilerParams(
            dimension_semantics=("parallel",)),
    )(h, q, W3, b3.reshape(1, D))


def kernel(x, edge_index, W1, b1, W2, b2, W3, b3):
    edges = edge_index.astype(jnp.int32).reshape(2, NW, NCHUNK, CH, BLK)
    zeros = jnp.zeros((N, D), jnp.float32)

    p = _sc_aggregate(x, zeros, edges)
    h = _mlp1(x, p, W1, b1, W2, b2)
    q = _sc_aggregate(h, zeros, edges)
    return _mlp2(h, q, W3, b3)


# speedup vs baseline: 1.0559x; 1.0559x over previous
import functools

import jax
import jax.numpy as jnp
from jax import lax
from jax.experimental import pallas as pl
from jax.experimental.pallas import tpu as pltpu
from jax.experimental.pallas import tpu_sc as plsc

N = 10000
E = 320000
D = 128

NC = 2
NS = 16
NW = NC * NS
BLK = 80
WBLK = E // (NW * BLK)
CH = 25
NCHUNK = WBLK // CH


def _sc_aggregate(values, edges):
    mesh = plsc.VectorSubcoreMesh(core_axis_name="c", subcore_axis_name="s")

    @functools.partial(
        pl.kernel,
        out_type=jax.ShapeDtypeStruct((NC, N, D), jnp.float32),
        mesh=mesh,
        scratch_types=[
            pltpu.VMEM((CH, BLK), jnp.int32),
            pltpu.VMEM((CH, BLK), jnp.int32),
            pltpu.VMEM((BLK, D), jnp.float32),
            pltpu.VMEM((BLK, D), jnp.float32),
            pltpu.VMEM((BLK, D), jnp.float32),
            pltpu.VMEM_SHARED((N, D), jnp.float32),
            pltpu.SemaphoreType.DMA,
            pltpu.SemaphoreType.DMA,
            pltpu.SemaphoreType.DMA,
            pltpu.SemaphoreType.DMA,
            pltpu.SemaphoreType.DMA,
            pltpu.SemaphoreType.DMA,
        ],
    )
    def agg_kernel(x_hbm, e_hbm, out_hbm,
                   src_v, dst_v, rows0, rows1, rows2, acc_sh,
                   g0, g1, g2, s0, s1, s2):
        cid = lax.axis_index("c")
        sid = lax.axis_index("s")
        wid = sid * NC + cid

        zv = jnp.zeros((16,), jnp.float32)

        @pl.loop(0, BLK)
        def _(r):
            for q in range(0, D, 16):
                rows0[r, pl.ds(q, 16)] = zv

        off = pl.multiple_of(sid * 624, 8)

        @pl.when(sid < 15)
        def _():
            for t in range(7):
                pltpu.sync_copy(rows0, acc_sh.at[pl.ds(off + 80 * t, 80)])
            pltpu.sync_copy(rows0.at[pl.ds(0, 64)],
                            acc_sh.at[pl.ds(off + 560, 64)])

        @pl.when(sid == 15)
        def _():
            for t in range(8):
                pltpu.sync_copy(rows0, acc_sh.at[pl.ds(9360 + 80 * t, 80)])

        plsc.subcore_barrier()

        rows = (rows0, rows1, rows2)
        gsem = (g0, g1, g2)
        ssem = (s0, s1, s2)
        gwait = [pltpu.make_async_copy(x_hbm.at[src_v.at[0]], rows[k], gsem[k])
                 for k in range(3)]
        swait = [pltpu.make_async_copy(rows[k], acc_sh.at[dst_v.at[0]],
                                       ssem[k]) for k in range(3)]

        def gath(b, k):
            pltpu.async_copy(x_hbm.at[src_v.at[b]], rows[k], gsem[k])

        def scat(b, k):
            pltpu.async_copy(rows[k], acc_sh.at[dst_v.at[b]], ssem[k],
                             add=True)

        @pl.loop(0, NCHUNK)
        def _(c):
            pltpu.sync_copy(e_hbm.at[0, wid, c], src_v)
            pltpu.sync_copy(e_hbm.at[1, wid, c], dst_v)
            gath(0, 0)
            gath(1, 1)
            gwait[0].wait()
            scat(0, 0)
            gath(2, 2)

            @pl.loop(0, (CH - 4) // 3)
            def _(m):
                j = 3 * m + 1
                gwait[1].wait()
                scat(j, 1)
                swait[0].wait()
                gath(j + 2, 0)
                gwait[2].wait()
                scat(j + 1, 2)
                swait[1].wait()
                gath(j + 3, 1)
                gwait[0].wait()
                scat(j + 2, 0)
                swait[2].wait()
                gath(j + 4, 2)

            gwait[1].wait()
            scat(CH - 3, 1)
            swait[0].wait()
            gath(CH - 1, 0)
            gwait[2].wait()
            scat(CH - 2, 2)
            gwait[0].wait()
            scat(CH - 1, 0)
            swait[1].wait()
            swait[2].wait()
            swait[0].wait()

        plsc.subcore_barrier()

        @pl.when(sid < 15)
        def _():
            pltpu.sync_copy(acc_sh.at[pl.ds(off, 624)],
                            out_hbm.at[cid, pl.ds(off, 624)])

        @pl.when(sid == 15)
        def _():
            pltpu.sync_copy(acc_sh.at[pl.ds(9360, 640)],
                            out_hbm.at[cid, pl.ds(9360, 640)])

    return agg_kernel(values, edges)


ROW_BLK = 2000


def _mlp1_body(x_ref, p_ref, w1_ref, b1_ref, w2_ref, b2_ref, o_ref):
    h = x_ref[...] + p_ref[0] + p_ref[1]
    a = lax.dot_general(h, w1_ref[...], (((1,), (0,)), ((), ())),
                        precision=lax.Precision.DEFAULT,
                        preferred_element_type=jnp.float32)
    a = jnp.maximum(a + b1_ref[...], 0.0)
    hh = lax.dot_general(a, w2_ref[...], (((1,), (0,)), ((), ())),
                         precision=lax.Precision.DEFAULT,
                         preferred_element_type=jnp.float32)
    hh = hh + b2_ref[...]
    o_ref[...] = jnp.where(hh > 0, hh, jnp.exp(hh) - 1.0)


def _mlp2_body(h_ref, q_ref, w3_ref, b3_ref, o_ref):
    h2 = h_ref[...] + q_ref[0] + q_ref[1]
    a = lax.dot_general(h2, w3_ref[...], (((1,), (0,)), ((), ())),
                        precision=lax.Precision.DEFAULT,
                        preferred_element_type=jnp.float32)
    o_ref[...] = jnp.maximum(a + b3_ref[...], 0.0)


def _row_spec():
    return pl.BlockSpec((ROW_BLK, D), lambda i: (i, 0))


def _pair_spec():
    return pl.BlockSpec((NC, ROW_BLK, D), lambda i: (0, i, 0))


def _full_spec(shape):
    return pl.BlockSpec(shape, lambda i: tuple(0 for _ in shape))


def _mlp1(x, p, W1, b1, W2, b2):
    return pl.pallas_call(
        _mlp1_body,
        grid=(N // ROW_BLK,),
        in_specs=[_row_spec(), _pair_spec(),
                  _full_spec((D, D)), _full_spec((1, D)),
                  _full_spec((D, D)), _full_spec((1, D))],
        out_specs=_row_spec(),
        out_shape=jax.ShapeDtypeStruct((N, D), jnp.float32),
        compiler_params=pltpu.CompilerParams(
            dimension_semantics=("parallel",)),
    )(x, p, W1, b1.reshape(1, D), W2, b2.reshape(1, D))


def _mlp2(h, q, W3, b3):
    return pl.pallas_call(
        _mlp2_body,
        grid=(N // ROW_BLK,),
        in_specs=[_row_spec(), _pair_spec(),
                  _full_spec((D, D)), _full_spec((1, D))],
        out_specs=_row_spec(),
        out_shape=jax.ShapeDtypeStruct((N, D), jnp.float32),
        compiler_params=pltpu.CompilerParams(
            dimension_semantics=("parallel",)),
    )(h, q, W3, b3.reshape(1, D))


def kernel(x, edge_index, W1, b1, W2, b2, W3, b3):
    edges = edge_index.astype(jnp.int32).reshape(2, NW, NCHUNK, CH, BLK)
    p = _sc_aggregate(x, edges)
    h = _mlp1(x, p, W1, b1, W2, b2)
    q = _sc_aggregate(h, edges)
    return _mlp2(h, q, W3, b3)


# speedup vs baseline: 1.1307x; 1.0709x over previous
import functools

import jax
import jax.numpy as jnp
from jax import lax
from jax.experimental import pallas as pl
from jax.experimental.pallas import tpu as pltpu
from jax.experimental.pallas import tpu_sc as plsc

N = 10000
E = 320000
D = 128

NC = 2
NS = 16
NW = NC * NS
BLK = 80
WBLK = E // (NW * BLK)
CH = 25
NCHUNK = WBLK // CH


def _sc_aggregate(values, edges):
    mesh = plsc.VectorSubcoreMesh(core_axis_name="c", subcore_axis_name="s")

    @functools.partial(
        pl.kernel,
        out_type=jax.ShapeDtypeStruct((NC, N, D), jnp.float32),
        mesh=mesh,
        scratch_types=[
            pltpu.VMEM((CH, BLK), jnp.int32),
            pltpu.VMEM((CH, BLK), jnp.int32),
            pltpu.VMEM((CH, BLK), jnp.int32),
            pltpu.VMEM((CH, BLK), jnp.int32),
            pltpu.VMEM((BLK, D), jnp.float32),
            pltpu.VMEM((BLK, D), jnp.float32),
            pltpu.VMEM((BLK, D), jnp.float32),
            pltpu.VMEM_SHARED((N, D), jnp.float32),
            pltpu.SemaphoreType.DMA,
            pltpu.SemaphoreType.DMA,
            pltpu.SemaphoreType.DMA,
            pltpu.SemaphoreType.DMA,
            pltpu.SemaphoreType.DMA,
            pltpu.SemaphoreType.DMA,
            pltpu.SemaphoreType.DMA,
            pltpu.SemaphoreType.DMA,
        ],
    )
    def agg_kernel(x_hbm, e_hbm, out_hbm,
                   sv0, dv0, sv1, dv1, rows0, rows1, rows2, acc_sh,
                   g0, g1, g2, s0, s1, s2, e0, e1):
        cid = lax.axis_index("c")
        sid = lax.axis_index("s")
        wid = sid * NC + cid

        zv = jnp.zeros((16,), jnp.float32)

        @pl.loop(0, BLK)
        def _(r):
            for q in range(0, D, 16):
                rows0[r, pl.ds(q, 16)] = zv

        off = pl.multiple_of(sid * 624, 8)

        @pl.when(sid < 15)
        def _():
            for t in range(7):
                pltpu.sync_copy(rows0, acc_sh.at[pl.ds(off + 80 * t, 80)])
            pltpu.sync_copy(rows0.at[pl.ds(0, 64)],
                            acc_sh.at[pl.ds(off + 560, 64)])

        @pl.when(sid == 15)
        def _():
            for t in range(8):
                pltpu.sync_copy(rows0, acc_sh.at[pl.ds(9360 + 80 * t, 80)])

        plsc.subcore_barrier()

        rows = (rows0, rows1, rows2)
        gsem = (g0, g1, g2)
        ssem = (s0, s1, s2)
        sv = (sv0, sv1)
        dv = (dv0, dv1)
        esem = (e0, e1)
        gwait = [pltpu.make_async_copy(x_hbm.at[sv0.at[0]], rows[k], gsem[k])
                 for k in range(3)]
        swait = [pltpu.make_async_copy(rows[k], acc_sh.at[dv0.at[0]],
                                       ssem[k]) for k in range(3)]
        eswait = [pltpu.make_async_copy(e_hbm.at[0, wid, 0], sv[t], esem[t])
                  for t in range(2)]
        edwait = [pltpu.make_async_copy(e_hbm.at[1, wid, 0], dv[t], esem[t])
                  for t in range(2)]

        def gath(slab, b, k):
            pltpu.async_copy(x_hbm.at[slab.at[b]], rows[k], gsem[k])

        def scat(slab, b, k):
            pltpu.async_copy(rows[k], acc_sh.at[slab.at[b]], ssem[k],
                             add=True)

        pltpu.sync_copy(e_hbm.at[0, wid, 0], sv[0])
        pltpu.sync_copy(e_hbm.at[1, wid, 0], dv[0])
        pltpu.async_copy(e_hbm.at[0, wid, 1], sv[1], esem[1])
        pltpu.async_copy(e_hbm.at[1, wid, 1], dv[1], esem[1])
        gath(sv[0], 0, 0)
        gath(sv[0], 1, 1)

        for c in range(NCHUNK):
            cur, nxt = c % 2, (c + 1) % 2
            rt = c % 3
            svc, dvc = sv[cur], dv[cur]
            svn, dvn = sv[nxt], dv[nxt]

            def bf(b, rt=rt):
                return (b + rt) % 3

            gwait[bf(0)].wait()
            scat(dvc, 0, bf(0))
            if c > 0:
                swait[(rt + 2) % 3].wait()
                if c < NCHUNK - 1:
                    pltpu.async_copy(e_hbm.at[0, wid, c + 1], svn, esem[nxt])
                    pltpu.async_copy(e_hbm.at[1, wid, c + 1], dvn, esem[nxt])
            gath(svc, 2, bf(2))

            k0, k1, k2 = bf(0), bf(1), bf(2)

            @pl.loop(0, 7)
            def _(m, svc=svc, dvc=dvc, k0=k0, k1=k1, k2=k2):
                j = 3 * m + 1
                gwait[k1].wait()
                scat(dvc, j, k1)
                swait[k0].wait()
                gath(svc, j + 2, k0)
                gwait[k2].wait()
                scat(dvc, j + 1, k2)
                swait[k1].wait()
                gath(svc, j + 3, k1)
                gwait[k0].wait()
                scat(dvc, j + 2, k0)
                swait[k2].wait()
                gath(svc, j + 4, k2)

            gwait[bf(22)].wait()
            scat(dvc, 22, bf(22))
            swait[bf(21)].wait()
            gath(svc, 24, bf(24))
            gwait[bf(23)].wait()
            scat(dvc, 23, bf(23))
            swait[bf(22)].wait()
            if c < NCHUNK - 1:
                eswait[nxt].wait()
                edwait[nxt].wait()
                gath(svn, 0, bf(25))
            gwait[bf(24)].wait()
            scat(dvc, 24, bf(24))
            swait[bf(23)].wait()
            if c < NCHUNK - 1:
                gath(svn, 1, bf(26))

        swait[(24 + NCHUNK - 1) % 3].wait()

        plsc.subcore_barrier()

        @pl.when(sid < 15)
        def _():
            pltpu.sync_copy(acc_sh.at[pl.ds(off, 624)],
                            out_hbm.at[cid, pl.ds(off, 624)])

        @pl.when(sid == 15)
        def _():
            pltpu.sync_copy(acc_sh.at[pl.ds(9360, 640)],
                            out_hbm.at[cid, pl.ds(9360, 640)])

    return agg_kernel(values, edges)


ROW_BLK = 2000


def _mlp1_body(x_ref, p_ref, w1_ref, b1_ref, w2_ref, b2_ref, o_ref):
    h = x_ref[...] + p_ref[0] + p_ref[1]
    a = lax.dot_general(h, w1_ref[...], (((1,), (0,)), ((), ())),
                        precision=lax.Precision.DEFAULT,
                        preferred_element_type=jnp.float32)
    a = jnp.maximum(a + b1_ref[...], 0.0)
    hh = lax.dot_general(a, w2_ref[...], (((1,), (0,)), ((), ())),
                         precision=lax.Precision.DEFAULT,
                         preferred_element_type=jnp.float32)
    hh = hh + b2_ref[...]
    o_ref[...] = jnp.where(hh > 0, hh, jnp.exp(hh) - 1.0)


def _mlp2_body(h_ref, q_ref, w3_ref, b3_ref, o_ref):
    h2 = h_ref[...] + q_ref[0] + q_ref[1]
    a = lax.dot_general(h2, w3_ref[...], (((1,), (0,)), ((), ())),
                        precision=lax.Precision.DEFAULT,
                        preferred_element_type=jnp.float32)
    o_ref[...] = jnp.maximum(a + b3_ref[...], 0.0)


def _row_spec():
    return pl.BlockSpec((ROW_BLK, D), lambda i: (i, 0))


def _pair_spec():
    return pl.BlockSpec((NC, ROW_BLK, D), lambda i: (0, i, 0))


def _full_spec(shape):
    return pl.BlockSpec(shape, lambda i: tuple(0 for _ in shape))


def _mlp1(x, p, W1, b1, W2, b2):
    return pl.pallas_call(
        _mlp1_body,
        grid=(N // ROW_BLK,),
        in_specs=[_row_spec(), _pair_spec(),
                  _full_spec((D, D)), _full_spec((1, D)),
                  _full_spec((D, D)), _full_spec((1, D))],
        out_specs=_row_spec(),
        out_shape=jax.ShapeDtypeStruct((N, D), jnp.float32),
        compiler_params=pltpu.CompilerParams(
            dimension_semantics=("parallel",)),
    )(x, p, W1, b1.reshape(1, D), W2, b2.reshape(1, D))


def _mlp2(h, q, W3, b3):
    return pl.pallas_call(
        _mlp2_body,
        grid=(N // ROW_BLK,),
        in_specs=[_row_spec(), _pair_spec(),
                  _full_spec((D, D)), _full_spec((1, D))],
        out_specs=_row_spec(),
        out_shape=jax.ShapeDtypeStruct((N, D), jnp.float32),
        compiler_params=pltpu.CompilerParams(
            dimension_semantics=("parallel",)),
    )(h, q, W3, b3.reshape(1, D))


def kernel(x, edge_index, W1, b1, W2, b2, W3, b3):
    edges = edge_index.astype(jnp.int32).reshape(2, NW, NCHUNK, CH, BLK)
    p = _sc_aggregate(x, edges)
    h = _mlp1(x, p, W1, b1, W2, b2)
    q = _sc_aggregate(h, edges)
    return _mlp2(h, q, W3, b3)
